# baseline (speedup 1.0000x reference)
import functools

import jax
import jax.numpy as jnp
from jax import lax
from jax.experimental import pallas as pl
from jax.experimental.pallas import tpu as pltpu
from jax.experimental.pallas import tpu_sc as plsc

N = 10000
E = 160000
G = 64
H = 512
DC = 128
NSUB = 16
EPAD = 163840
EPT = EPAD // NSUB
KIDX = 64
NGRP = EPT // KIDX
NBLK = 10
NRING = 4
ACC_ROWS = 10000
DRAIN = 640
TR = 1000


def _sc_seg_sum(C):
    mesh = plsc.VectorSubcoreMesh(core_axis_name="c", subcore_axis_name="s",
                                  num_cores=2)
    CHW = C // 2

    @functools.partial(
        pl.kernel,
        out_type=jax.ShapeDtypeStruct((C, N, DC), jnp.float32),
        mesh=mesh,
        scratch_types=[
            pltpu.VMEM((NBLK, KIDX), jnp.int32),
            pltpu.VMEM((NBLK, KIDX), jnp.int32),
            pltpu.VMEM((NRING, KIDX, DC), jnp.float32),
            pltpu.VMEM_SHARED((ACC_ROWS, DC), jnp.float32),
            pltpu.SemaphoreType.DMA,
            pltpu.SemaphoreType.DMA,
            pltpu.SemaphoreType.DMA,
            pltpu.SemaphoreType.DMA,
        ],
    )
    def seg_sum(h_hbm, src_hbm, dst_hbm, zero_hbm, out_hbm,
                sidx, didx, rows, acc, sem0, sem1, sem2, sem3):
        c = lax.axis_index("c")
        s = lax.axis_index("s")
        sems = [sem0, sem1, sem2, sem3]

        for k in range(CHW):
            cc = 2 * k + c
            @pl.when(s < 15)
            def _zero():
                pltpu.sync_copy(zero_hbm, acc.at[pl.ds(s * DRAIN, DRAIN)])

            @pl.when(s == 15)
            def _zero_tail():
                pltpu.sync_copy(
                    zero_hbm.at[pl.ds(0, N - 15 * DRAIN)],
                    acc.at[pl.ds(15 * DRAIN, N - 15 * DRAIN)])

            plsc.subcore_barrier()

            @pl.loop(0, NGRP // NBLK)
            def _blk(b):
                pltpu.sync_copy(src_hbm.at[b].at[s], sidx)
                pltpu.sync_copy(dst_hbm.at[b].at[s], didx)
                cps = [None] * NRING
                for g in range(NRING - 1):
                    cps[g] = pltpu.async_copy(
                        h_hbm.at[cc].at[sidx.at[g]], rows.at[g], sems[g])
                for g in range(NBLK):
                    sl = g % NRING
                    if g + NRING - 1 < NBLK:
                        ng = g + NRING - 1
                        nsl = ng % NRING
                        cps[nsl] = pltpu.async_copy(
                            h_hbm.at[cc].at[sidx.at[ng]],
                            rows.at[nsl], sems[nsl])
                    cps[sl].wait()
                    pltpu.sync_copy(rows.at[sl],
                                    acc.at[didx.at[g]], add=True)

            plsc.subcore_barrier()

            @pl.when(s < 15)
            def _drain():
                pltpu.sync_copy(
                    acc.at[pl.ds(s * DRAIN, DRAIN)],
                    out_hbm.at[cc].at[pl.ds(s * DRAIN, DRAIN)])

            @pl.when(s == 15)
            def _drain_tail():
                pltpu.sync_copy(
                    acc.at[pl.ds(15 * DRAIN, N - 15 * DRAIN)],
                    out_hbm.at[cc].at[pl.ds(15 * DRAIN, N - 15 * DRAIN)])

            plsc.subcore_barrier()

    return seg_sum


def _tc_mlp(C, din):
    nt = N // TR

    def body(h_ref, a_ref, w1_ref, b1_ref, w2_ref, b2_ref, m_ref, st_ref):
        i = pl.program_id(0)
        h = jnp.concatenate([h_ref[cc] for cc in range(C)], axis=-1)
        ag = jnp.concatenate([a_ref[cc] for cc in range(C)], axis=-1)
        m = h + ag
        corr = jnp.where(
            (lax.broadcasted_iota(jnp.int32, (TR, 1), 0) == 0) & (i == 0),
            float(EPAD - E), 0.0)
        m = m - corr * h[0:1, :]
        p = jnp.dot(m, w1_ref[...], preferred_element_type=jnp.float32)
        p = jnp.maximum(p + b1_ref[...], 0.0)
        q = jnp.dot(p, w2_ref[...], preferred_element_type=jnp.float32)
        q = jnp.maximum(q + b2_ref[...], 0.0)
        m_ref[...] = q
        st = jnp.concatenate(
            [jnp.sum(q, axis=0, keepdims=True),
             jnp.sum(q * q, axis=0, keepdims=True)], axis=0)

        @pl.when(i == 0)
        def _():
            st_ref[...] = jnp.zeros_like(st_ref)

        st_ref[...] += st

    return pl.pallas_call(
        body,
        grid=(nt,),
        in_specs=[
            pl.BlockSpec((C, TR, DC), lambda i: (0, i, 0)),
            pl.BlockSpec((C, TR, DC), lambda i: (0, i, 0)),
            pl.BlockSpec((din, H), lambda i: (0, 0)),
            pl.BlockSpec((1, H), lambda i: (0, 0)),
            pl.BlockSpec((H, H), lambda i: (0, 0)),
            pl.BlockSpec((1, H), lambda i: (0, 0)),
        ],
        out_specs=[
            pl.BlockSpec((TR, H), lambda i: (i, 0)),
            pl.BlockSpec((2, H), lambda i: (0, 0)),
        ],
        out_shape=[
            jax.ShapeDtypeStruct((N, H), jnp.float32),
            jax.ShapeDtypeStruct((2, H), jnp.float32),
        ],
    )


def _tc_bn_pool():
    nt = N // TR
    C = H // DC

    def body(m_ref, st_ref, g_ref, be_ref, b3_ref, h_ref, pool_ref):
        i = pl.program_id(0)
        q = m_ref[...]
        mu = st_ref[0:1, :] / N
        var = st_ref[1:2, :] / N - mu * mu
        inv = lax.rsqrt(var + 1e-5) * g_ref[...]
        hb = (q - mu) * inv + be_ref[...]
        for cc in range(C):
            h_ref[cc] = hb[:, cc * DC:(cc + 1) * DC]
        b = b3_ref[0]
        oh = (lax.broadcasted_iota(jnp.int32, (G, TR), 0) == b
              ).astype(jnp.float32)
        pool = jnp.dot(oh, hb, preferred_element_type=jnp.float32)

        @pl.when(i == 0)
        def _():
            pool_ref[...] = jnp.zeros_like(pool_ref)

        pool_ref[...] += pool

    return pl.pallas_call(
        body,
        grid=(nt,),
        in_specs=[
            pl.BlockSpec((TR, H), lambda i: (i, 0)),
            pl.BlockSpec((2, H), lambda i: (0, 0)),
            pl.BlockSpec((1, H), lambda i: (0, 0)),
            pl.BlockSpec((1, H), lambda i: (0, 0)),
            pl.BlockSpec((1, 1, TR), lambda i: (i, 0, 0)),
        ],
        out_specs=[
            pl.BlockSpec((C, TR, DC), lambda i: (0, i, 0)),
            pl.BlockSpec((G, H), lambda i: (0, 0)),
        ],
        out_shape=[
            jax.ShapeDtypeStruct((C, N, DC), jnp.float32),
            jax.ShapeDtypeStruct((G, H), jnp.float32),
        ],
    )


def _tc_proj():
    P = 3 * H

    def body(p0, p1, p2, wp1, bp1, gp, bep, a_ref, wp2, bp2, out_ref):
        xp = jnp.concatenate([p0[...], p1[...], p2[...]], axis=1)
        z = jnp.dot(xp, wp1[...], preferred_element_type=jnp.float32)
        z = z + bp1[...]
        mu = jnp.mean(z, axis=0, keepdims=True)
        var = jnp.mean(z * z, axis=0, keepdims=True) - mu * mu
        z = (z - mu) * lax.rsqrt(var + 1e-5) * gp[...] + bep[...]
        a = a_ref[0, 0]
        z = jnp.where(z >= 0, z, a * z)
        out_ref[...] = jnp.dot(z, wp2[...],
                               preferred_element_type=jnp.float32) + bp2[...]

    return pl.pallas_call(
        body,
        out_shape=jax.ShapeDtypeStruct((G, P), jnp.float32),
    )


def kernel(x, edge_index, batch, device,
           W1_0, b1_0, W2_0, b2_0, g_0, be_0,
           W1_1, b1_1, W2_1, b2_1, g_1, be_1,
           W1_2, b1_2, W2_2, b2_2, g_2, be_2,
           Wp1, bp1, gp, bep, a, Wp2, bp2):
    f32 = jnp.float32
    c0 = x.shape[1] // DC
    hc = jnp.transpose(x.reshape(N, c0, DC), (1, 0, 2))

    pad = EPAD - E
    srcp = jnp.concatenate([edge_index[0], jnp.zeros((pad,), jnp.int32)])
    dstp = jnp.concatenate([edge_index[1], jnp.zeros((pad,), jnp.int32)])
    src3 = jnp.transpose(
        srcp.reshape(NSUB, NGRP // NBLK, NBLK, KIDX), (1, 0, 2, 3))
    dst3 = jnp.transpose(
        dstp.reshape(NSUB, NGRP // NBLK, NBLK, KIDX), (1, 0, 2, 3))
    zer = jnp.zeros((DRAIN, DC), f32)
    batch3 = batch.reshape(N // TR, 1, TR)

    layers = [
        (W1_0, b1_0, W2_0, b2_0, g_0, be_0),
        (W1_1, b1_1, W2_1, b2_1, g_1, be_1),
        (W1_2, b1_2, W2_2, b2_2, g_2, be_2),
    ]
    pools = []
    for (W1, b1, W2, b2, g, be) in layers:
        C = hc.shape[0]
        agg = _sc_seg_sum(C)(hc, src3, dst3, zer)
        m2, st = _tc_mlp(C, C * DC)(hc, agg, W1, b1.reshape(1, H),
                                    W2, b2.reshape(1, H))
        hc, pool = _tc_bn_pool()(m2, st, g.reshape(1, H), be.reshape(1, H),
                                 batch3)
        pools.append(pool)

    a2 = jnp.broadcast_to(a.reshape(1, 1), (1, 128)).astype(f32)
    out = _tc_proj()(pools[0], pools[1], pools[2],
                     Wp1, bp1.reshape(1, 3 * H), gp.reshape(1, 3 * H),
                     bep.reshape(1, 3 * H), a2, Wp2, bp2.reshape(1, 3 * H))
    return out


# speedup vs baseline: 1.0014x; 1.0014x over previous
import functools

import jax
import jax.numpy as jnp
from jax import lax
from jax.experimental import pallas as pl
from jax.experimental.pallas import tpu as pltpu
from jax.experimental.pallas import tpu_sc as plsc

N = 10000
E = 160000
G = 64
H = 512
DC = 128
NSUB = 16
EPAD = 163840
EPT = EPAD // NSUB
KIDX = 64
NGRP = EPT // KIDX
NBLK = 10
NRING = 4
ACC_ROWS = 10000
DRAIN = 640
TR = 1000


def _sc_seg_sum(C):
    mesh = plsc.VectorSubcoreMesh(core_axis_name="c", subcore_axis_name="s",
                                  num_cores=2)
    CHW = C // 2

    @functools.partial(
        pl.kernel,
        out_type=jax.ShapeDtypeStruct((C, N, DC), jnp.float32),
        mesh=mesh,
        scratch_types=[
            pltpu.VMEM((NBLK, KIDX), jnp.int32),
            pltpu.VMEM((NBLK, KIDX), jnp.int32),
            pltpu.VMEM((NRING, KIDX, DC), jnp.float32),
            pltpu.VMEM_SHARED((ACC_ROWS, DC), jnp.float32),
            pltpu.SemaphoreType.DMA,
            pltpu.SemaphoreType.DMA,
            pltpu.SemaphoreType.DMA,
            pltpu.SemaphoreType.DMA,
            pltpu.SemaphoreType.DMA,
            pltpu.SemaphoreType.DMA,
            pltpu.SemaphoreType.DMA,
            pltpu.SemaphoreType.DMA,
        ],
    )
    def seg_sum(h_hbm, src_hbm, dst_hbm, zero_hbm, out_hbm,
                sidx, didx, rows, acc,
                sem0, sem1, sem2, sem3, asem0, asem1, asem2, asem3):
        c = lax.axis_index("c")
        s = lax.axis_index("s")
        sems = [sem0, sem1, sem2, sem3]
        asems = [asem0, asem1, asem2, asem3]

        for k in range(CHW):
            cc = 2 * k + c
            @pl.when(s < 15)
            def _zero():
                pltpu.sync_copy(zero_hbm, acc.at[pl.ds(s * DRAIN, DRAIN)])

            @pl.when(s == 15)
            def _zero_tail():
                pltpu.sync_copy(
                    zero_hbm.at[pl.ds(0, N - 15 * DRAIN)],
                    acc.at[pl.ds(15 * DRAIN, N - 15 * DRAIN)])

            plsc.subcore_barrier()

            @pl.loop(0, NGRP // NBLK)
            def _blk(b):
                pltpu.sync_copy(src_hbm.at[b].at[s], sidx)
                pltpu.sync_copy(dst_hbm.at[b].at[s], didx)
                cps = [None] * NRING
                adds = [None] * NRING
                for g in range(NRING - 1):
                    cps[g] = pltpu.async_copy(
                        h_hbm.at[cc].at[sidx.at[g]], rows.at[g], sems[g])
                for g in range(NBLK):
                    sl = g % NRING
                    if g + NRING - 1 < NBLK:
                        ng = g + NRING - 1
                        nsl = ng % NRING
                        if adds[nsl] is not None:
                            adds[nsl].wait()
                            adds[nsl] = None
                        cps[nsl] = pltpu.async_copy(
                            h_hbm.at[cc].at[sidx.at[ng]],
                            rows.at[nsl], sems[nsl])
                    cps[sl].wait()
                    adds[sl] = pltpu.async_copy(
                        rows.at[sl], acc.at[didx.at[g]], asems[sl],
                        add=True)
                for sl in range(NRING):
                    if adds[sl] is not None:
                        adds[sl].wait()

            plsc.subcore_barrier()

            @pl.when(s < 15)
            def _drain():
                pltpu.sync_copy(
                    acc.at[pl.ds(s * DRAIN, DRAIN)],
                    out_hbm.at[cc].at[pl.ds(s * DRAIN, DRAIN)])

            @pl.when(s == 15)
            def _drain_tail():
                pltpu.sync_copy(
                    acc.at[pl.ds(15 * DRAIN, N - 15 * DRAIN)],
                    out_hbm.at[cc].at[pl.ds(15 * DRAIN, N - 15 * DRAIN)])

            plsc.subcore_barrier()

    return seg_sum


def _tc_mlp(C, din):
    nt = N // TR

    def body(h_ref, a_ref, w1_ref, b1_ref, w2_ref, b2_ref, m_ref, st_ref):
        i = pl.program_id(0)
        h = jnp.concatenate([h_ref[cc] for cc in range(C)], axis=-1)
        ag = jnp.concatenate([a_ref[cc] for cc in range(C)], axis=-1)
        m = h + ag
        corr = jnp.where(
            (lax.broadcasted_iota(jnp.int32, (TR, 1), 0) == 0) & (i == 0),
            float(EPAD - E), 0.0)
        m = m - corr * h[0:1, :]
        p = jnp.dot(m, w1_ref[...], preferred_element_type=jnp.float32)
        p = jnp.maximum(p + b1_ref[...], 0.0)
        q = jnp.dot(p, w2_ref[...], preferred_element_type=jnp.float32)
        q = jnp.maximum(q + b2_ref[...], 0.0)
        m_ref[...] = q
        st = jnp.concatenate(
            [jnp.sum(q, axis=0, keepdims=True),
             jnp.sum(q * q, axis=0, keepdims=True)], axis=0)

        @pl.when(i == 0)
        def _():
            st_ref[...] = jnp.zeros_like(st_ref)

        st_ref[...] += st

    return pl.pallas_call(
        body,
        grid=(nt,),
        in_specs=[
            pl.BlockSpec((C, TR, DC), lambda i: (0, i, 0)),
            pl.BlockSpec((C, TR, DC), lambda i: (0, i, 0)),
            pl.BlockSpec((din, H), lambda i: (0, 0)),
            pl.BlockSpec((1, H), lambda i: (0, 0)),
            pl.BlockSpec((H, H), lambda i: (0, 0)),
            pl.BlockSpec((1, H), lambda i: (0, 0)),
        ],
        out_specs=[
            pl.BlockSpec((TR, H), lambda i: (i, 0)),
            pl.BlockSpec((2, H), lambda i: (0, 0)),
        ],
        out_shape=[
            jax.ShapeDtypeStruct((N, H), jnp.float32),
            jax.ShapeDtypeStruct((2, H), jnp.float32),
        ],
    )


def _tc_bn_pool():
    nt = N // TR
    C = H // DC

    def body(m_ref, st_ref, g_ref, be_ref, b3_ref, h_ref, pool_ref):
        i = pl.program_id(0)
        q = m_ref[...]
        mu = st_ref[0:1, :] / N
        var = st_ref[1:2, :] / N - mu * mu
        inv = lax.rsqrt(var + 1e-5) * g_ref[...]
        hb = (q - mu) * inv + be_ref[...]
        for cc in range(C):
            h_ref[cc] = hb[:, cc * DC:(cc + 1) * DC]
        b = b3_ref[0]
        oh = (lax.broadcasted_iota(jnp.int32, (G, TR), 0) == b
              ).astype(jnp.float32)
        pool = jnp.dot(oh, hb, preferred_element_type=jnp.float32)

        @pl.when(i == 0)
        def _():
            pool_ref[...] = jnp.zeros_like(pool_ref)

        pool_ref[...] += pool

    return pl.pallas_call(
        body,
        grid=(nt,),
        in_specs=[
            pl.BlockSpec((TR, H), lambda i: (i, 0)),
            pl.BlockSpec((2, H), lambda i: (0, 0)),
            pl.BlockSpec((1, H), lambda i: (0, 0)),
            pl.BlockSpec((1, H), lambda i: (0, 0)),
            pl.BlockSpec((1, 1, TR), lambda i: (i, 0, 0)),
        ],
        out_specs=[
            pl.BlockSpec((C, TR, DC), lambda i: (0, i, 0)),
            pl.BlockSpec((G, H), lambda i: (0, 0)),
        ],
        out_shape=[
            jax.ShapeDtypeStruct((C, N, DC), jnp.float32),
            jax.ShapeDtypeStruct((G, H), jnp.float32),
        ],
    )


def _tc_proj():
    P = 3 * H

    def body(p0, p1, p2, wp1, bp1, gp, bep, a_ref, wp2, bp2, out_ref):
        xp = jnp.concatenate([p0[...], p1[...], p2[...]], axis=1)
        z = jnp.dot(xp, wp1[...], preferred_element_type=jnp.float32)
        z = z + bp1[...]
        mu = jnp.mean(z, axis=0, keepdims=True)
        var = jnp.mean(z * z, axis=0, keepdims=True) - mu * mu
        z = (z - mu) * lax.rsqrt(var + 1e-5) * gp[...] + bep[...]
        a = a_ref[0, 0]
        z = jnp.where(z >= 0, z, a * z)
        out_ref[...] = jnp.dot(z, wp2[...],
                               preferred_element_type=jnp.float32) + bp2[...]

    return pl.pallas_call(
        body,
        out_shape=jax.ShapeDtypeStruct((G, P), jnp.float32),
    )


def kernel(x, edge_index, batch, device,
           W1_0, b1_0, W2_0, b2_0, g_0, be_0,
           W1_1, b1_1, W2_1, b2_1, g_1, be_1,
           W1_2, b1_2, W2_2, b2_2, g_2, be_2,
           Wp1, bp1, gp, bep, a, Wp2, bp2):
    f32 = jnp.float32
    c0 = x.shape[1] // DC
    hc = jnp.transpose(x.reshape(N, c0, DC), (1, 0, 2))

    pad = EPAD - E
    srcp = jnp.concatenate([edge_index[0], jnp.zeros((pad,), jnp.int32)])
    dstp = jnp.concatenate([edge_index[1], jnp.zeros((pad,), jnp.int32)])
    src3 = jnp.transpose(
        srcp.reshape(NSUB, NGRP // NBLK, NBLK, KIDX), (1, 0, 2, 3))
    dst3 = jnp.transpose(
        dstp.reshape(NSUB, NGRP // NBLK, NBLK, KIDX), (1, 0, 2, 3))
    zer = jnp.zeros((DRAIN, DC), f32)
    batch3 = batch.reshape(N // TR, 1, TR)

    layers = [
        (W1_0, b1_0, W2_0, b2_0, g_0, be_0),
        (W1_1, b1_1, W2_1, b2_1, g_1, be_1),
        (W1_2, b1_2, W2_2, b2_2, g_2, be_2),
    ]
    pools = []
    for (W1, b1, W2, b2, g, be) in layers:
        C = hc.shape[0]
        agg = _sc_seg_sum(C)(hc, src3, dst3, zer)
        m2, st = _tc_mlp(C, C * DC)(hc, agg, W1, b1.reshape(1, H),
                                    W2, b2.reshape(1, H))
        hc, pool = _tc_bn_pool()(m2, st, g.reshape(1, H), be.reshape(1, H),
                                 batch3)
        pools.append(pool)

    a2 = jnp.broadcast_to(a.reshape(1, 1), (1, 128)).astype(f32)
    out = _tc_proj()(pools[0], pools[1], pools[2],
                     Wp1, bp1.reshape(1, 3 * H), gp.reshape(1, 3 * H),
                     bep.reshape(1, 3 * H), a2, Wp2, bp2.reshape(1, 3 * H))
    return out


# speedup vs baseline: 1.0256x; 1.0242x over previous
import functools

import jax
import jax.numpy as jnp
from jax import lax
from jax.experimental import pallas as pl
from jax.experimental.pallas import tpu as pltpu
from jax.experimental.pallas import tpu_sc as plsc

N = 10000
E = 160000
G = 64
H = 512
DC = 128
NSUB = 16
EPAD = 163840
EPT = EPAD // NSUB
KIDX = 128
NGRP = EPT // KIDX
NBLK = 8
NRING = 2
ACC_ROWS = 10000
DRAIN = 640
TR = 1000


def _sc_seg_sum(C):
    mesh = plsc.VectorSubcoreMesh(core_axis_name="c", subcore_axis_name="s",
                                  num_cores=2)
    CHW = C // 2

    @functools.partial(
        pl.kernel,
        out_type=jax.ShapeDtypeStruct((C, N, DC), jnp.float32),
        mesh=mesh,
        scratch_types=[
            pltpu.VMEM((NBLK, KIDX), jnp.int32),
            pltpu.VMEM((NBLK, KIDX), jnp.int32),
            pltpu.VMEM((NRING, KIDX, DC), jnp.float32),
            pltpu.VMEM_SHARED((ACC_ROWS, DC), jnp.float32),
            pltpu.SemaphoreType.DMA,
            pltpu.SemaphoreType.DMA,
            pltpu.SemaphoreType.DMA,
            pltpu.SemaphoreType.DMA,
            pltpu.SemaphoreType.DMA,
            pltpu.SemaphoreType.DMA,
            pltpu.SemaphoreType.DMA,
            pltpu.SemaphoreType.DMA,
        ],
    )
    def seg_sum(h_hbm, src_hbm, dst_hbm, zero_hbm, out_hbm,
                sidx, didx, rows, acc,
                sem0, sem1, sem2, sem3, asem0, asem1, asem2, asem3):
        c = lax.axis_index("c")
        s = lax.axis_index("s")
        sems = [sem0, sem1, sem2, sem3]
        asems = [asem0, asem1, asem2, asem3]

        for k in range(CHW):
            cc = 2 * k + c
            @pl.when(s < 15)
            def _zero():
                pltpu.sync_copy(zero_hbm, acc.at[pl.ds(s * DRAIN, DRAIN)])

            @pl.when(s == 15)
            def _zero_tail():
                pltpu.sync_copy(
                    zero_hbm.at[pl.ds(0, N - 15 * DRAIN)],
                    acc.at[pl.ds(15 * DRAIN, N - 15 * DRAIN)])

            plsc.subcore_barrier()

            @pl.loop(0, NGRP // NBLK)
            def _blk(b):
                pltpu.sync_copy(src_hbm.at[b].at[s], sidx)
                pltpu.sync_copy(dst_hbm.at[b].at[s], didx)
                cps = [None] * NRING
                adds = [None] * NRING
                for g in range(NRING - 1):
                    cps[g] = pltpu.async_copy(
                        h_hbm.at[cc].at[sidx.at[g]], rows.at[g], sems[g])
                for g in range(NBLK):
                    sl = g % NRING
                    if g + NRING - 1 < NBLK:
                        ng = g + NRING - 1
                        nsl = ng % NRING
                        if adds[nsl] is not None:
                            adds[nsl].wait()
                            adds[nsl] = None
                        cps[nsl] = pltpu.async_copy(
                            h_hbm.at[cc].at[sidx.at[ng]],
                            rows.at[nsl], sems[nsl])
                    cps[sl].wait()
                    adds[sl] = pltpu.async_copy(
                        rows.at[sl], acc.at[didx.at[g]], asems[sl],
                        add=True)
                for sl in range(NRING):
                    if adds[sl] is not None:
                        adds[sl].wait()

            plsc.subcore_barrier()

            @pl.when(s < 15)
            def _drain():
                pltpu.sync_copy(
                    acc.at[pl.ds(s * DRAIN, DRAIN)],
                    out_hbm.at[cc].at[pl.ds(s * DRAIN, DRAIN)])

            @pl.when(s == 15)
            def _drain_tail():
                pltpu.sync_copy(
                    acc.at[pl.ds(15 * DRAIN, N - 15 * DRAIN)],
                    out_hbm.at[cc].at[pl.ds(15 * DRAIN, N - 15 * DRAIN)])

            plsc.subcore_barrier()

    return seg_sum


def _tc_mlp(C, din):
    nt = N // TR

    def body(h_ref, a_ref, w1_ref, b1_ref, w2_ref, b2_ref, m_ref, st_ref):
        i = pl.program_id(0)
        h = jnp.concatenate([h_ref[cc] for cc in range(C)], axis=-1)
        ag = jnp.concatenate([a_ref[cc] for cc in range(C)], axis=-1)
        m = h + ag
        corr = jnp.where(
            (lax.broadcasted_iota(jnp.int32, (TR, 1), 0) == 0) & (i == 0),
            float(EPAD - E), 0.0)
        m = m - corr * h[0:1, :]
        p = jnp.dot(m, w1_ref[...], preferred_element_type=jnp.float32)
        p = jnp.maximum(p + b1_ref[...], 0.0)
        q = jnp.dot(p, w2_ref[...], preferred_element_type=jnp.float32)
        q = jnp.maximum(q + b2_ref[...], 0.0)
        m_ref[...] = q
        st = jnp.concatenate(
            [jnp.sum(q, axis=0, keepdims=True),
             jnp.sum(q * q, axis=0, keepdims=True)], axis=0)

        @pl.when(i == 0)
        def _():
            st_ref[...] = jnp.zeros_like(st_ref)

        st_ref[...] += st

    return pl.pallas_call(
        body,
        grid=(nt,),
        in_specs=[
            pl.BlockSpec((C, TR, DC), lambda i: (0, i, 0)),
            pl.BlockSpec((C, TR, DC), lambda i: (0, i, 0)),
            pl.BlockSpec((din, H), lambda i: (0, 0)),
            pl.BlockSpec((1, H), lambda i: (0, 0)),
            pl.BlockSpec((H, H), lambda i: (0, 0)),
            pl.BlockSpec((1, H), lambda i: (0, 0)),
        ],
        out_specs=[
            pl.BlockSpec((TR, H), lambda i: (i, 0)),
            pl.BlockSpec((2, H), lambda i: (0, 0)),
        ],
        out_shape=[
            jax.ShapeDtypeStruct((N, H), jnp.float32),
            jax.ShapeDtypeStruct((2, H), jnp.float32),
        ],
    )


def _tc_bn_pool():
    nt = N // TR
    C = H // DC

    def body(m_ref, st_ref, g_ref, be_ref, b3_ref, h_ref, pool_ref):
        i = pl.program_id(0)
        q = m_ref[...]
        mu = st_ref[0:1, :] / N
        var = st_ref[1:2, :] / N - mu * mu
        inv = lax.rsqrt(var + 1e-5) * g_ref[...]
        hb = (q - mu) * inv + be_ref[...]
        for cc in range(C):
            h_ref[cc] = hb[:, cc * DC:(cc + 1) * DC]
        b = b3_ref[0]
        oh = (lax.broadcasted_iota(jnp.int32, (G, TR), 0) == b
              ).astype(jnp.float32)
        pool = jnp.dot(oh, hb, preferred_element_type=jnp.float32)

        @pl.when(i == 0)
        def _():
            pool_ref[...] = jnp.zeros_like(pool_ref)

        pool_ref[...] += pool

    return pl.pallas_call(
        body,
        grid=(nt,),
        in_specs=[
            pl.BlockSpec((TR, H), lambda i: (i, 0)),
            pl.BlockSpec((2, H), lambda i: (0, 0)),
            pl.BlockSpec((1, H), lambda i: (0, 0)),
            pl.BlockSpec((1, H), lambda i: (0, 0)),
            pl.BlockSpec((1, 1, TR), lambda i: (i, 0, 0)),
        ],
        out_specs=[
            pl.BlockSpec((C, TR, DC), lambda i: (0, i, 0)),
            pl.BlockSpec((G, H), lambda i: (0, 0)),
        ],
        out_shape=[
            jax.ShapeDtypeStruct((C, N, DC), jnp.float32),
            jax.ShapeDtypeStruct((G, H), jnp.float32),
        ],
    )


def _tc_proj():
    P = 3 * H

    def body(p0, p1, p2, wp1, bp1, gp, bep, a_ref, wp2, bp2, out_ref):
        xp = jnp.concatenate([p0[...], p1[...], p2[...]], axis=1)
        z = jnp.dot(xp, wp1[...], preferred_element_type=jnp.float32)
        z = z + bp1[...]
        mu = jnp.mean(z, axis=0, keepdims=True)
        var = jnp.mean(z * z, axis=0, keepdims=True) - mu * mu
        z = (z - mu) * lax.rsqrt(var + 1e-5) * gp[...] + bep[...]
        a = a_ref[0, 0]
        z = jnp.where(z >= 0, z, a * z)
        out_ref[...] = jnp.dot(z, wp2[...],
                               preferred_element_type=jnp.float32) + bp2[...]

    return pl.pallas_call(
        body,
        out_shape=jax.ShapeDtypeStruct((G, P), jnp.float32),
    )


def kernel(x, edge_index, batch, device,
           W1_0, b1_0, W2_0, b2_0, g_0, be_0,
           W1_1, b1_1, W2_1, b2_1, g_1, be_1,
           W1_2, b1_2, W2_2, b2_2, g_2, be_2,
           Wp1, bp1, gp, bep, a, Wp2, bp2):
    f32 = jnp.float32
    c0 = x.shape[1] // DC
    hc = jnp.transpose(x.reshape(N, c0, DC), (1, 0, 2))

    pad = EPAD - E
    srcp = jnp.concatenate([edge_index[0], jnp.zeros((pad,), jnp.int32)])
    dstp = jnp.concatenate([edge_index[1], jnp.zeros((pad,), jnp.int32)])
    src3 = jnp.transpose(
        srcp.reshape(NSUB, NGRP // NBLK, NBLK, KIDX), (1, 0, 2, 3))
    dst3 = jnp.transpose(
        dstp.reshape(NSUB, NGRP // NBLK, NBLK, KIDX), (1, 0, 2, 3))
    zer = jnp.zeros((DRAIN, DC), f32)
    batch3 = batch.reshape(N // TR, 1, TR)

    layers = [
        (W1_0, b1_0, W2_0, b2_0, g_0, be_0),
        (W1_1, b1_1, W2_1, b2_1, g_1, be_1),
        (W1_2, b1_2, W2_2, b2_2, g_2, be_2),
    ]
    pools = []
    for (W1, b1, W2, b2, g, be) in layers:
        C = hc.shape[0]
        agg = _sc_seg_sum(C)(hc, src3, dst3, zer)
        m2, st = _tc_mlp(C, C * DC)(hc, agg, W1, b1.reshape(1, H),
                                    W2, b2.reshape(1, H))
        hc, pool = _tc_bn_pool()(m2, st, g.reshape(1, H), be.reshape(1, H),
                                 batch3)
        pools.append(pool)

    a2 = jnp.broadcast_to(a.reshape(1, 1), (1, 128)).astype(f32)
    out = _tc_proj()(pools[0], pools[1], pools[2],
                     Wp1, bp1.reshape(1, 3 * H), gp.reshape(1, 3 * H),
                     bep.reshape(1, 3 * H), a2, Wp2, bp2.reshape(1, 3 * H))
    return out


# speedup vs baseline: 1.0524x; 1.0261x over previous
import functools

import jax
import jax.numpy as jnp
from jax import lax
from jax.experimental import pallas as pl
from jax.experimental.pallas import tpu as pltpu
from jax.experimental.pallas import tpu_sc as plsc

N = 10000
E = 160000
G = 64
H = 512
DC = 128
NSUB = 16
EPAD = 163840
EPT = EPAD // NSUB
KIDX = 128
NGRP = EPT // KIDX
NBLK = 10
NRING = 2
ACC_ROWS = 10000
DRAIN = 640
TR = 1000


def _sc_seg_sum(C):
    mesh = plsc.VectorSubcoreMesh(core_axis_name="c", subcore_axis_name="s",
                                  num_cores=2)
    CHW = C // 2

    @functools.partial(
        pl.kernel,
        out_type=jax.ShapeDtypeStruct((C, N, DC), jnp.float32),
        mesh=mesh,
        scratch_types=[
            pltpu.VMEM((NBLK, KIDX), jnp.int32),
            pltpu.VMEM((NBLK, KIDX), jnp.int32),
            pltpu.VMEM((NRING, KIDX, DC), jnp.float32),
            pltpu.VMEM_SHARED((ACC_ROWS, DC), jnp.float32),
            pltpu.SemaphoreType.DMA,
            pltpu.SemaphoreType.DMA,
            pltpu.SemaphoreType.DMA,
            pltpu.SemaphoreType.DMA,
            pltpu.SemaphoreType.DMA,
            pltpu.SemaphoreType.DMA,
            pltpu.SemaphoreType.DMA,
            pltpu.SemaphoreType.DMA,
        ],
    )
    def seg_sum(h_hbm, src_hbm, dst_hbm, zero_hbm, out_hbm,
                sidx, didx, rows, acc,
                sem0, sem1, sem2, sem3, asem0, asem1, asem2, asem3):
        c = lax.axis_index("c")
        s = lax.axis_index("s")
        sems = [sem0, sem1, sem2, sem3]
        asems = [asem0, asem1, asem2, asem3]

        for k in range(CHW):
            cc = 2 * k + c
            @pl.when(s < 15)
            def _zero():
                pltpu.sync_copy(zero_hbm, acc.at[pl.ds(s * DRAIN, DRAIN)])

            @pl.when(s == 15)
            def _zero_tail():
                pltpu.sync_copy(
                    zero_hbm.at[pl.ds(0, N - 15 * DRAIN)],
                    acc.at[pl.ds(15 * DRAIN, N - 15 * DRAIN)])

            plsc.subcore_barrier()

            @pl.loop(0, NGRP // NBLK)
            def _blk(b):
                pltpu.sync_copy(src_hbm.at[b].at[s], sidx)
                pltpu.sync_copy(dst_hbm.at[b].at[s], didx)
                cps = [None] * NRING
                adds = [None] * NRING
                for g in range(NRING - 1):
                    cps[g] = pltpu.async_copy(
                        h_hbm.at[cc].at[sidx.at[g]], rows.at[g], sems[g])
                for g in range(NBLK):
                    sl = g % NRING
                    if g + NRING - 1 < NBLK:
                        ng = g + NRING - 1
                        nsl = ng % NRING
                        if adds[nsl] is not None:
                            adds[nsl].wait()
                            adds[nsl] = None
                        cps[nsl] = pltpu.async_copy(
                            h_hbm.at[cc].at[sidx.at[ng]],
                            rows.at[nsl], sems[nsl])
                    cps[sl].wait()
                    adds[sl] = pltpu.async_copy(
                        rows.at[sl], acc.at[didx.at[g]], asems[sl],
                        add=True)
                for sl in range(NRING):
                    if adds[sl] is not None:
                        adds[sl].wait()

            plsc.subcore_barrier()

            @pl.when(s < 15)
            def _drain():
                pltpu.sync_copy(
                    acc.at[pl.ds(s * DRAIN, DRAIN)],
                    out_hbm.at[cc].at[pl.ds(s * DRAIN, DRAIN)])

            @pl.when(s == 15)
            def _drain_tail():
                pltpu.sync_copy(
                    acc.at[pl.ds(15 * DRAIN, N - 15 * DRAIN)],
                    out_hbm.at[cc].at[pl.ds(15 * DRAIN, N - 15 * DRAIN)])

            plsc.subcore_barrier()

    return seg_sum


def _tc_mlp(C, din):
    nt = N // TR

    def body(h_ref, a_ref, w1_ref, b1_ref, w2_ref, b2_ref, m_ref, st_ref):
        i = pl.program_id(0)
        h = jnp.concatenate([h_ref[cc] for cc in range(C)], axis=-1)
        ag = jnp.concatenate([a_ref[cc] for cc in range(C)], axis=-1)
        m = h + ag
        corr = jnp.where(
            (lax.broadcasted_iota(jnp.int32, (TR, 1), 0) == 0) & (i == 0),
            float(EPAD - E), 0.0)
        m = m - corr * h[0:1, :]
        p = jnp.dot(m, w1_ref[...], preferred_element_type=jnp.float32)
        p = jnp.maximum(p + b1_ref[...], 0.0)
        q = jnp.dot(p, w2_ref[...], preferred_element_type=jnp.float32)
        q = jnp.maximum(q + b2_ref[...], 0.0)
        m_ref[...] = q
        st = jnp.concatenate(
            [jnp.sum(q, axis=0, keepdims=True),
             jnp.sum(q * q, axis=0, keepdims=True)], axis=0)

        @pl.when(i == 0)
        def _():
            st_ref[...] = jnp.zeros_like(st_ref)

        st_ref[...] += st

    return pl.pallas_call(
        body,
        grid=(nt,),
        in_specs=[
            pl.BlockSpec((C, TR, DC), lambda i: (0, i, 0)),
            pl.BlockSpec((C, TR, DC), lambda i: (0, i, 0)),
            pl.BlockSpec((din, H), lambda i: (0, 0)),
            pl.BlockSpec((1, H), lambda i: (0, 0)),
            pl.BlockSpec((H, H), lambda i: (0, 0)),
            pl.BlockSpec((1, H), lambda i: (0, 0)),
        ],
        out_specs=[
            pl.BlockSpec((TR, H), lambda i: (i, 0)),
            pl.BlockSpec((2, H), lambda i: (0, 0)),
        ],
        out_shape=[
            jax.ShapeDtypeStruct((N, H), jnp.float32),
            jax.ShapeDtypeStruct((2, H), jnp.float32),
        ],
    )


def _tc_bn_pool():
    nt = N // TR
    C = H // DC

    def body(m_ref, st_ref, g_ref, be_ref, b3_ref, h_ref, pool_ref):
        i = pl.program_id(0)
        q = m_ref[...]
        mu = st_ref[0:1, :] / N
        var = st_ref[1:2, :] / N - mu * mu
        inv = lax.rsqrt(var + 1e-5) * g_ref[...]
        hb = (q - mu) * inv + be_ref[...]
        for cc in range(C):
            h_ref[cc] = hb[:, cc * DC:(cc + 1) * DC]
        b = b3_ref[0]
        oh = (lax.broadcasted_iota(jnp.int32, (G, TR), 0) == b
              ).astype(jnp.float32)
        pool = jnp.dot(oh, hb, preferred_element_type=jnp.float32)

        @pl.when(i == 0)
        def _():
            pool_ref[...] = jnp.zeros_like(pool_ref)

        pool_ref[...] += pool

    return pl.pallas_call(
        body,
        grid=(nt,),
        in_specs=[
            pl.BlockSpec((TR, H), lambda i: (i, 0)),
            pl.BlockSpec((2, H), lambda i: (0, 0)),
            pl.BlockSpec((1, H), lambda i: (0, 0)),
            pl.BlockSpec((1, H), lambda i: (0, 0)),
            pl.BlockSpec((1, 1, TR), lambda i: (i, 0, 0)),
        ],
        out_specs=[
            pl.BlockSpec((C, TR, DC), lambda i: (0, i, 0)),
            pl.BlockSpec((G, H), lambda i: (0, 0)),
        ],
        out_shape=[
            jax.ShapeDtypeStruct((C, N, DC), jnp.float32),
            jax.ShapeDtypeStruct((G, H), jnp.float32),
        ],
    )


def _tc_proj():
    P = 3 * H

    def body(p0, p1, p2, wp1, bp1, gp, bep, a_ref, wp2, bp2, out_ref):
        xp = jnp.concatenate([p0[...], p1[...], p2[...]], axis=1)
        z = jnp.dot(xp, wp1[...], preferred_element_type=jnp.float32)
        z = z + bp1[...]
        mu = jnp.mean(z, axis=0, keepdims=True)
        var = jnp.mean(z * z, axis=0, keepdims=True) - mu * mu
        z = (z - mu) * lax.rsqrt(var + 1e-5) * gp[...] + bep[...]
        a = a_ref[0, 0]
        z = jnp.where(z >= 0, z, a * z)
        out_ref[...] = jnp.dot(z, wp2[...],
                               preferred_element_type=jnp.float32) + bp2[...]

    return pl.pallas_call(
        body,
        out_shape=jax.ShapeDtypeStruct((G, P), jnp.float32),
    )


def kernel(x, edge_index, batch, device,
           W1_0, b1_0, W2_0, b2_0, g_0, be_0,
           W1_1, b1_1, W2_1, b2_1, g_1, be_1,
           W1_2, b1_2, W2_2, b2_2, g_2, be_2,
           Wp1, bp1, gp, bep, a, Wp2, bp2):
    f32 = jnp.float32
    c0 = x.shape[1] // DC
    hc = jnp.transpose(x.reshape(N, c0, DC), (1, 0, 2))

    pad = EPAD - E
    srcp = jnp.concatenate([edge_index[0], jnp.zeros((pad,), jnp.int32)])
    dstp = jnp.concatenate([edge_index[1], jnp.zeros((pad,), jnp.int32)])
    src3 = jnp.transpose(
        srcp.reshape(NSUB, NGRP // NBLK, NBLK, KIDX), (1, 0, 2, 3))
    dst3 = jnp.transpose(
        dstp.reshape(NSUB, NGRP // NBLK, NBLK, KIDX), (1, 0, 2, 3))
    zer = jnp.zeros((DRAIN, DC), f32)
    batch3 = batch.reshape(N // TR, 1, TR)

    layers = [
        (W1_0, b1_0, W2_0, b2_0, g_0, be_0),
        (W1_1, b1_1, W2_1, b2_1, g_1, be_1),
        (W1_2, b1_2, W2_2, b2_2, g_2, be_2),
    ]
    pools = []
    for (W1, b1, W2, b2, g, be) in layers:
        C = hc.shape[0]
        agg = _sc_seg_sum(C)(hc, src3, dst3, zer)
        m2, st = _tc_mlp(C, C * DC)(hc, agg, W1, b1.reshape(1, H),
                                    W2, b2.reshape(1, H))
        hc, pool = _tc_bn_pool()(m2, st, g.reshape(1, H), be.reshape(1, H),
                                 batch3)
        pools.append(pool)

    a2 = jnp.broadcast_to(a.reshape(1, 1), (1, 128)).astype(f32)
    out = _tc_proj()(pools[0], pools[1], pools[2],
                     Wp1, bp1.reshape(1, 3 * H), gp.reshape(1, 3 * H),
                     bep.reshape(1, 3 * H), a2, Wp2, bp2.reshape(1, 3 * H))
    return out


# speedup vs baseline: 1.0585x; 1.0058x over previous
import functools

import jax
import jax.numpy as jnp
from jax import lax
from jax.experimental import pallas as pl
from jax.experimental.pallas import tpu as pltpu
from jax.experimental.pallas import tpu_sc as plsc

N = 10000
E = 160000
G = 64
H = 512
DC = 128
NSUB = 16
EPAD = 163840
EPT = EPAD // NSUB
KIDX = 128
NGRP = EPT // KIDX
NBLK = 16
NRING = 2
ACC_ROWS = 10000
DRAIN = 640
TR = 1000


def _sc_seg_sum(C):
    mesh = plsc.VectorSubcoreMesh(core_axis_name="c", subcore_axis_name="s",
                                  num_cores=2)
    CHW = C // 2

    @functools.partial(
        pl.kernel,
        out_type=jax.ShapeDtypeStruct((C, N, DC), jnp.float32),
        mesh=mesh,
        scratch_types=[
            pltpu.VMEM((NBLK, KIDX), jnp.int32),
            pltpu.VMEM((NBLK, KIDX), jnp.int32),
            pltpu.VMEM((NRING, KIDX, DC), jnp.float32),
            pltpu.VMEM_SHARED((ACC_ROWS, DC), jnp.float32),
            pltpu.SemaphoreType.DMA,
            pltpu.SemaphoreType.DMA,
            pltpu.SemaphoreType.DMA,
            pltpu.SemaphoreType.DMA,
            pltpu.SemaphoreType.DMA,
            pltpu.SemaphoreType.DMA,
            pltpu.SemaphoreType.DMA,
            pltpu.SemaphoreType.DMA,
        ],
    )
    def seg_sum(h_hbm, src_hbm, dst_hbm, zero_hbm, out_hbm,
                sidx, didx, rows, acc,
                sem0, sem1, sem2, sem3, asem0, asem1, asem2, asem3):
        c = lax.axis_index("c")
        s = lax.axis_index("s")
        sems = [sem0, sem1, sem2, sem3]
        asems = [asem0, asem1, asem2, asem3]

        for k in range(CHW):
            cc = 2 * k + c
            @pl.when(s < 15)
            def _zero():
                pltpu.sync_copy(zero_hbm, acc.at[pl.ds(s * DRAIN, DRAIN)])

            @pl.when(s == 15)
            def _zero_tail():
                pltpu.sync_copy(
                    zero_hbm.at[pl.ds(0, N - 15 * DRAIN)],
                    acc.at[pl.ds(15 * DRAIN, N - 15 * DRAIN)])

            plsc.subcore_barrier()

            @pl.loop(0, NGRP // NBLK)
            def _blk(b):
                pltpu.sync_copy(src_hbm.at[b].at[s], sidx)
                pltpu.sync_copy(dst_hbm.at[b].at[s], didx)
                cps = [None] * NRING
                adds = [None] * NRING
                for g in range(NRING - 1):
                    cps[g] = pltpu.async_copy(
                        h_hbm.at[cc].at[sidx.at[g]], rows.at[g], sems[g])
                for g in range(NBLK):
                    sl = g % NRING
                    if g + NRING - 1 < NBLK:
                        ng = g + NRING - 1
                        nsl = ng % NRING
                        if adds[nsl] is not None:
                            adds[nsl].wait()
                            adds[nsl] = None
                        cps[nsl] = pltpu.async_copy(
                            h_hbm.at[cc].at[sidx.at[ng]],
                            rows.at[nsl], sems[nsl])
                    cps[sl].wait()
                    adds[sl] = pltpu.async_copy(
                        rows.at[sl], acc.at[didx.at[g]], asems[sl],
                        add=True)
                for sl in range(NRING):
                    if adds[sl] is not None:
                        adds[sl].wait()

            plsc.subcore_barrier()

            @pl.when(s < 15)
            def _drain():
                pltpu.sync_copy(
                    acc.at[pl.ds(s * DRAIN, DRAIN)],
                    out_hbm.at[cc].at[pl.ds(s * DRAIN, DRAIN)])

            @pl.when(s == 15)
            def _drain_tail():
                pltpu.sync_copy(
                    acc.at[pl.ds(15 * DRAIN, N - 15 * DRAIN)],
                    out_hbm.at[cc].at[pl.ds(15 * DRAIN, N - 15 * DRAIN)])

            plsc.subcore_barrier()

    return seg_sum


def _tc_mlp(C, din):
    nt = N // TR

    def body(h_ref, a_ref, w1_ref, b1_ref, w2_ref, b2_ref, m_ref, st_ref):
        i = pl.program_id(0)
        h = jnp.concatenate([h_ref[cc] for cc in range(C)], axis=-1)
        ag = jnp.concatenate([a_ref[cc] for cc in range(C)], axis=-1)
        m = h + ag
        corr = jnp.where(
            (lax.broadcasted_iota(jnp.int32, (TR, 1), 0) == 0) & (i == 0),
            float(EPAD - E), 0.0)
        m = m - corr * h[0:1, :]
        p = jnp.dot(m, w1_ref[...], preferred_element_type=jnp.float32)
        p = jnp.maximum(p + b1_ref[...], 0.0)
        q = jnp.dot(p, w2_ref[...], preferred_element_type=jnp.float32)
        q = jnp.maximum(q + b2_ref[...], 0.0)
        m_ref[...] = q
        st = jnp.concatenate(
            [jnp.sum(q, axis=0, keepdims=True),
             jnp.sum(q * q, axis=0, keepdims=True)], axis=0)

        @pl.when(i == 0)
        def _():
            st_ref[...] = jnp.zeros_like(st_ref)

        st_ref[...] += st

    return pl.pallas_call(
        body,
        grid=(nt,),
        in_specs=[
            pl.BlockSpec((C, TR, DC), lambda i: (0, i, 0)),
            pl.BlockSpec((C, TR, DC), lambda i: (0, i, 0)),
            pl.BlockSpec((din, H), lambda i: (0, 0)),
            pl.BlockSpec((1, H), lambda i: (0, 0)),
            pl.BlockSpec((H, H), lambda i: (0, 0)),
            pl.BlockSpec((1, H), lambda i: (0, 0)),
        ],
        out_specs=[
            pl.BlockSpec((TR, H), lambda i: (i, 0)),
            pl.BlockSpec((2, H), lambda i: (0, 0)),
        ],
        out_shape=[
            jax.ShapeDtypeStruct((N, H), jnp.float32),
            jax.ShapeDtypeStruct((2, H), jnp.float32),
        ],
    )


def _tc_bn_pool():
    nt = N // TR
    C = H // DC

    def body(m_ref, st_ref, g_ref, be_ref, b3_ref, h_ref, pool_ref):
        i = pl.program_id(0)
        q = m_ref[...]
        mu = st_ref[0:1, :] / N
        var = st_ref[1:2, :] / N - mu * mu
        inv = lax.rsqrt(var + 1e-5) * g_ref[...]
        hb = (q - mu) * inv + be_ref[...]
        for cc in range(C):
            h_ref[cc] = hb[:, cc * DC:(cc + 1) * DC]
        b = b3_ref[0]
        oh = (lax.broadcasted_iota(jnp.int32, (G, TR), 0) == b
              ).astype(jnp.float32)
        pool = jnp.dot(oh, hb, preferred_element_type=jnp.float32)

        @pl.when(i == 0)
        def _():
            pool_ref[...] = jnp.zeros_like(pool_ref)

        pool_ref[...] += pool

    return pl.pallas_call(
        body,
        grid=(nt,),
        in_specs=[
            pl.BlockSpec((TR, H), lambda i: (i, 0)),
            pl.BlockSpec((2, H), lambda i: (0, 0)),
            pl.BlockSpec((1, H), lambda i: (0, 0)),
            pl.BlockSpec((1, H), lambda i: (0, 0)),
            pl.BlockSpec((1, 1, TR), lambda i: (i, 0, 0)),
        ],
        out_specs=[
            pl.BlockSpec((C, TR, DC), lambda i: (0, i, 0)),
            pl.BlockSpec((G, H), lambda i: (0, 0)),
        ],
        out_shape=[
            jax.ShapeDtypeStruct((C, N, DC), jnp.float32),
            jax.ShapeDtypeStruct((G, H), jnp.float32),
        ],
    )


def _tc_proj():
    P = 3 * H

    def body(p0, p1, p2, wp1, bp1, gp, bep, a_ref, wp2, bp2, out_ref):
        xp = jnp.concatenate([p0[...], p1[...], p2[...]], axis=1)
        z = jnp.dot(xp, wp1[...], preferred_element_type=jnp.float32)
        z = z + bp1[...]
        mu = jnp.mean(z, axis=0, keepdims=True)
        var = jnp.mean(z * z, axis=0, keepdims=True) - mu * mu
        z = (z - mu) * lax.rsqrt(var + 1e-5) * gp[...] + bep[...]
        a = a_ref[0, 0]
        z = jnp.where(z >= 0, z, a * z)
        out_ref[...] = jnp.dot(z, wp2[...],
                               preferred_element_type=jnp.float32) + bp2[...]

    return pl.pallas_call(
        body,
        out_shape=jax.ShapeDtypeStruct((G, P), jnp.float32),
    )


def kernel(x, edge_index, batch, device,
           W1_0, b1_0, W2_0, b2_0, g_0, be_0,
           W1_1, b1_1, W2_1, b2_1, g_1, be_1,
           W1_2, b1_2, W2_2, b2_2, g_2, be_2,
           Wp1, bp1, gp, bep, a, Wp2, bp2):
    f32 = jnp.float32
    c0 = x.shape[1] // DC
    hc = jnp.transpose(x.reshape(N, c0, DC), (1, 0, 2))

    pad = EPAD - E
    srcp = jnp.concatenate([edge_index[0], jnp.zeros((pad,), jnp.int32)])
    dstp = jnp.concatenate([edge_index[1], jnp.zeros((pad,), jnp.int32)])
    src3 = jnp.transpose(
        srcp.reshape(NSUB, NGRP // NBLK, NBLK, KIDX), (1, 0, 2, 3))
    dst3 = jnp.transpose(
        dstp.reshape(NSUB, NGRP // NBLK, NBLK, KIDX), (1, 0, 2, 3))
    zer = jnp.zeros((DRAIN, DC), f32)
    batch3 = batch.reshape(N // TR, 1, TR)

    layers = [
        (W1_0, b1_0, W2_0, b2_0, g_0, be_0),
        (W1_1, b1_1, W2_1, b2_1, g_1, be_1),
        (W1_2, b1_2, W2_2, b2_2, g_2, be_2),
    ]
    pools = []
    for (W1, b1, W2, b2, g, be) in layers:
        C = hc.shape[0]
        agg = _sc_seg_sum(C)(hc, src3, dst3, zer)
        m2, st = _tc_mlp(C, C * DC)(hc, agg, W1, b1.reshape(1, H),
                                    W2, b2.reshape(1, H))
        hc, pool = _tc_bn_pool()(m2, st, g.reshape(1, H), be.reshape(1, H),
                                 batch3)
        pools.append(pool)

    a2 = jnp.broadcast_to(a.reshape(1, 1), (1, 128)).astype(f32)
    out = _tc_proj()(pools[0], pools[1], pools[2],
                     Wp1, bp1.reshape(1, 3 * H), gp.reshape(1, 3 * H),
                     bep.reshape(1, 3 * H), a2, Wp2, bp2.reshape(1, 3 * H))
    return out
